# baseline (speedup 1.0000x reference)
import functools

import jax
import jax.numpy as jnp
from jax import lax
from jax.experimental import pallas as pl
from jax.experimental.pallas import tpu as pltpu
from jax.experimental.pallas import tpu_sc as plsc

D = 32


def _sc_gather(user_id, item_id, Pf, Qf, Uf, Vf, n_rows):
    info = plsc.get_sparse_core_info()
    nw = info.num_cores * info.num_subcores
    nc = info.num_cores
    bsz = user_id.shape[0]
    b_per_w = bsz // nw
    nvec = b_per_w // 16
    lpw = b_per_w * D

    mesh = plsc.VectorSubcoreMesh(core_axis_name="c", subcore_axis_name="s")
    out_t = [jax.ShapeDtypeStruct((bsz * D,), jnp.float32) for _ in range(4)]

    @functools.partial(
        pl.kernel,
        mesh=mesh,
        out_type=out_t,
        scratch_types=[
            pltpu.VMEM((b_per_w,), jnp.int32),
            pltpu.VMEM((b_per_w,), jnp.int32),
            pltpu.VMEM((lpw,), jnp.int32),
            pltpu.VMEM((lpw,), jnp.int32),
            pltpu.VMEM((lpw,), jnp.float32),
            pltpu.VMEM((lpw,), jnp.float32),
            pltpu.VMEM((lpw,), jnp.float32),
            pltpu.VMEM((lpw,), jnp.float32),
            pltpu.SemaphoreType.DMA,
        ],
        compiler_params=pltpu.CompilerParams(use_tc_tiling_on_sc=False),
    )
    def gather_kernel(uid_h, iid_h, p_h, q_h, u_h, v_h,
                      po, qo, uo, vo,
                      ui_v, ii_v, uoff, ioff, pv, qv, uv, vv, sem):
        wid = lax.axis_index("s") * nc + lax.axis_index("c")
        base = wid * b_per_w
        pltpu.sync_copy(uid_h.at[pl.ds(base, b_per_w)], ui_v)
        pltpu.sync_copy(iid_h.at[pl.ds(base, b_per_w)], ii_v)

        def build(v, _):
            uvec = ui_v[pl.ds(v * 16, 16)]
            ivec = ii_v[pl.ds(v * 16, 16)]
            for j in range(D):
                uoff[pl.ds(j * b_per_w + v * 16, 16)] = uvec + j * n_rows
                ioff[pl.ds(j * b_per_w + v * 16, 16)] = ivec + j * n_rows
            return 0

        lax.fori_loop(0, nvec, build, 0)

        c1 = pltpu.async_copy(p_h.at[uoff], pv, sem)
        c2 = pltpu.async_copy(u_h.at[uoff], uv, sem)
        c3 = pltpu.async_copy(q_h.at[ioff], qv, sem)
        c4 = pltpu.async_copy(v_h.at[ioff], vv, sem)
        c1.wait()
        c2.wait()
        c3.wait()
        c4.wait()

        obase = wid * lpw
        pltpu.sync_copy(pv, po.at[pl.ds(obase, lpw)])
        pltpu.sync_copy(qv, qo.at[pl.ds(obase, lpw)])
        pltpu.sync_copy(uv, uo.at[pl.ds(obase, lpw)])
        pltpu.sync_copy(vv, vo.at[pl.ds(obase, lpw)])

    return gather_kernel(user_id, item_id, Pf, Qf, Uf, Vf)


def _tc_body(pmf_r, qmf_r, pmlp_r, qmlp_r,
             w1_r, b1_r, w2_r, b2_r, w3_r, b3_r, wp_r, out_r):
    h = jnp.concatenate([pmlp_r[...], qmlp_r[...]], axis=1)
    h = jnp.maximum(
        jnp.dot(h, w1_r[...], preferred_element_type=jnp.float32) + b1_r[...], 0.0)
    h = jnp.maximum(
        jnp.dot(h, w2_r[...], preferred_element_type=jnp.float32) + b2_r[...], 0.0)
    h = jnp.maximum(
        jnp.dot(h, w3_r[...], preferred_element_type=jnp.float32) + b3_r[...], 0.0)
    g = pmf_r[...] * qmf_r[...]
    z = jnp.concatenate([g, h], axis=1)
    out_r[...] = jnp.dot(z, wp_r[...], preferred_element_type=jnp.float32)


def _tc_dense(pmf, qmf, pmlp, qmlp, W1, b1, W2, b2, W3, b3, Wp):
    bsz = pmf.shape[0]
    blk = 2048
    grid = bsz // blk

    def row_spec():
        return pl.BlockSpec((blk, D), lambda i: (i, 0))

    def full_spec(shape):
        return pl.BlockSpec(shape, lambda i: tuple(0 for _ in shape))

    b1r = b1.reshape(1, -1)
    b2r = b2.reshape(1, -1)
    b3r = b3.reshape(1, -1)

    return pl.pallas_call(
        _tc_body,
        grid=(grid,),
        in_specs=[
            row_spec(), row_spec(), row_spec(), row_spec(),
            full_spec(W1.shape), full_spec(b1r.shape),
            full_spec(W2.shape), full_spec(b2r.shape),
            full_spec(W3.shape), full_spec(b3r.shape),
            full_spec(Wp.shape),
        ],
        out_specs=pl.BlockSpec((blk, 1), lambda i: (i, 0)),
        out_shape=jax.ShapeDtypeStruct((bsz, 1), jnp.float32),
    )(pmf, qmf, pmlp, qmlp, W1, b1r, W2, b2r, W3, b3r, Wp)


def _unshuffle(flat, nw, b_per_w):
    return flat.reshape(nw, D, b_per_w).transpose(0, 2, 1).reshape(-1, D)


def kernel(user_id, item_id, P, Q, U, V, W1, b1, W2, b2, W3, b3, Wp):
    uid = user_id.astype(jnp.int32)
    iid = item_id.astype(jnp.int32)
    info = plsc.get_sparse_core_info()
    nw = info.num_cores * info.num_subcores
    b_per_w = uid.shape[0] // nw
    n_rows = P.shape[0]
    pf, qf, uf, vf = _sc_gather(
        uid, iid, P.T.reshape(-1), Q.T.reshape(-1), U.T.reshape(-1),
        V.T.reshape(-1), n_rows)
    pmf = _unshuffle(pf, nw, b_per_w)
    qmf = _unshuffle(qf, nw, b_per_w)
    pmlp = _unshuffle(uf, nw, b_per_w)
    qmlp = _unshuffle(vf, nw, b_per_w)
    return _tc_dense(pmf, qmf, pmlp, qmlp, W1, b1, W2, b2, W3, b3, Wp)


# speedup vs baseline: 5.8022x; 5.8022x over previous
import functools

import jax
import jax.numpy as jnp
from jax import lax
from jax.experimental import pallas as pl
from jax.experimental.pallas import tpu as pltpu
from jax.experimental.pallas import tpu_sc as plsc

D = 32
PACK = 128 // D
CHUNK = 256


def _sc_gather(user_id, item_id, P4, Q4, U4, V4):
    info = plsc.get_sparse_core_info()
    nw = info.num_cores * info.num_subcores
    nc = info.num_cores
    bsz = user_id.shape[0]
    b_per_w = bsz // nw
    lpw = b_per_w * D
    n_chunks = b_per_w // CHUNK

    mesh = plsc.VectorSubcoreMesh(core_axis_name="c", subcore_axis_name="s")
    out_t = [jax.ShapeDtypeStruct((bsz * D,), jnp.float32) for _ in range(4)]

    @functools.partial(
        pl.kernel,
        mesh=mesh,
        out_type=out_t,
        scratch_types=[
            pltpu.VMEM((b_per_w,), jnp.int32),
            pltpu.VMEM((b_per_w,), jnp.int32),
            pltpu.VMEM((CHUNK,), jnp.int32),
            pltpu.VMEM((CHUNK, 128), jnp.float32),
            pltpu.VMEM((lpw,), jnp.float32),
            pltpu.VMEM((lpw,), jnp.float32),
            pltpu.VMEM((lpw,), jnp.float32),
            pltpu.VMEM((lpw,), jnp.float32),
            pltpu.SemaphoreType.DMA,
        ],
        compiler_params=pltpu.CompilerParams(needs_layout_passes=False),
    )
    def gather_kernel(uid_h, iid_h, p_h, q_h, u_h, v_h,
                      po, qo, uo, vo,
                      ui_v, ii_v, slice_idx, stage, pv, qv, uv, vv, sem):
        wid = lax.axis_index("s") * nc + lax.axis_index("c")
        base = wid * b_per_w
        pltpu.sync_copy(uid_h.at[pl.ds(base, b_per_w)], ui_v)
        pltpu.sync_copy(iid_h.at[pl.ds(base, b_per_w)], ii_v)

        lane = lax.iota(jnp.int32, 16)

        def do_table(t_h, idx_v, dst):
            def chunk_body(c, _):
                c0 = c * CHUNK

                def idx_body(v, _):
                    vec = idx_v[pl.ds(c0 + v * 16, 16)]
                    slice_idx[pl.ds(v * 16, 16)] = vec >> 2
                    return 0

                lax.fori_loop(0, CHUNK // 16, idx_body, 0)
                pltpu.async_copy(t_h.at[slice_idx], stage, sem).wait()

                def ext_body(v, _):
                    vec = idx_v[pl.ds(c0 + v * 16, 16)]
                    g32 = (vec & 3) * D
                    row = v * 16 + lane
                    for j in range(D):
                        vals = plsc.load_gather(stage, [row, g32 + j])
                        dst[pl.ds(j * b_per_w + c0 + v * 16, 16)] = vals
                    return 0

                lax.fori_loop(0, CHUNK // 16, ext_body, 0)
                return 0

            lax.fori_loop(0, n_chunks, chunk_body, 0)

        do_table(p_h, ui_v, pv)
        do_table(u_h, ui_v, uv)
        do_table(q_h, ii_v, qv)
        do_table(v_h, ii_v, vv)

        obase = wid * lpw
        pltpu.sync_copy(pv, po.at[pl.ds(obase, lpw)])
        pltpu.sync_copy(qv, qo.at[pl.ds(obase, lpw)])
        pltpu.sync_copy(uv, uo.at[pl.ds(obase, lpw)])
        pltpu.sync_copy(vv, vo.at[pl.ds(obase, lpw)])

    return gather_kernel(user_id, item_id, P4, Q4, U4, V4)


def _tc_body(pmf_r, qmf_r, pmlp_r, qmlp_r,
             w1_r, b1_r, w2_r, b2_r, w3_r, b3_r, wp_r, out_r):
    h = jnp.concatenate([pmlp_r[...], qmlp_r[...]], axis=1)
    h = jnp.maximum(
        jnp.dot(h, w1_r[...], preferred_element_type=jnp.float32) + b1_r[...], 0.0)
    h = jnp.maximum(
        jnp.dot(h, w2_r[...], preferred_element_type=jnp.float32) + b2_r[...], 0.0)
    h = jnp.maximum(
        jnp.dot(h, w3_r[...], preferred_element_type=jnp.float32) + b3_r[...], 0.0)
    g = pmf_r[...] * qmf_r[...]
    z = jnp.concatenate([g, h], axis=1)
    out_r[...] = jnp.dot(z, wp_r[...], preferred_element_type=jnp.float32)


def _tc_dense(pmf, qmf, pmlp, qmlp, W1, b1, W2, b2, W3, b3, Wp):
    bsz = pmf.shape[0]
    blk = 2048
    grid = bsz // blk

    def row_spec():
        return pl.BlockSpec((blk, D), lambda i: (i, 0))

    def full_spec(shape):
        return pl.BlockSpec(shape, lambda i: tuple(0 for _ in shape))

    b1r = b1.reshape(1, -1)
    b2r = b2.reshape(1, -1)
    b3r = b3.reshape(1, -1)

    return pl.pallas_call(
        _tc_body,
        grid=(grid,),
        in_specs=[
            row_spec(), row_spec(), row_spec(), row_spec(),
            full_spec(W1.shape), full_spec(b1r.shape),
            full_spec(W2.shape), full_spec(b2r.shape),
            full_spec(W3.shape), full_spec(b3r.shape),
            full_spec(Wp.shape),
        ],
        out_specs=pl.BlockSpec((blk, 1), lambda i: (i, 0)),
        out_shape=jax.ShapeDtypeStruct((bsz, 1), jnp.float32),
    )(pmf, qmf, pmlp, qmlp, W1, b1r, W2, b2r, W3, b3r, Wp)


def _unshuffle(flat, nw, b_per_w):
    return flat.reshape(nw, D, b_per_w).transpose(0, 2, 1).reshape(-1, D)


def kernel(user_id, item_id, P, Q, U, V, W1, b1, W2, b2, W3, b3, Wp):
    uid = user_id.astype(jnp.int32)
    iid = item_id.astype(jnp.int32)
    info = plsc.get_sparse_core_info()
    nw = info.num_cores * info.num_subcores
    b_per_w = uid.shape[0] // nw
    n4 = P.shape[0] // PACK
    pf, qf, uf, vf = _sc_gather(
        uid, iid, P.reshape(n4, 128), Q.reshape(n4, 128),
        U.reshape(n4, 128), V.reshape(n4, 128))
    pmf = _unshuffle(pf, nw, b_per_w)
    qmf = _unshuffle(qf, nw, b_per_w)
    pmlp = _unshuffle(uf, nw, b_per_w)
    qmlp = _unshuffle(vf, nw, b_per_w)
    return _tc_dense(pmf, qmf, pmlp, qmlp, W1, b1, W2, b2, W3, b3, Wp)
